# Initial kernel scaffold; baseline (speedup 1.0000x reference)
import jax
import jax.numpy as jnp
from jax.experimental import pallas as pl


def kernel(x, edge_attr, edge_index, batch, W_src, W_dst, att_src, att_dst, bias, ln_gamma, ln_beta, prelu_w):
    raise NotImplementedError("write your pallas kernel here")


# speedup vs baseline: 14.6556x; 14.6556x over previous
import functools

import jax
import jax.numpy as jnp
from jax import lax
from jax.experimental import pallas as pl
from jax.experimental.pallas import tpu as pltpu
from jax.experimental.pallas import tpu_sc as plsc

N = 10000
E = 160000
D = 256
C = 128
H = 4
HC = H * C

NC = 2
NS = 16
NW = NC * NS
NP = 10240
EP = 163840
EPW = EP // NW
ROWS_W = EPW // 128

@functools.cache
def _mesh():
    return plsc.VectorSubcoreMesh(
        core_axis_name="c", subcore_axis_name="s",
        num_cores=NC, num_subcores=NS)


_BN1 = 2000

def _k1_body(x_ref, ws_ref, wd_ref, ats_ref, atd_ref, xs_ref, as_ref, ad_ref):
    xb = x_ref[...]
    xs = lax.dot_general(xb, ws_ref[...], (((1,), (0,)), ((), ())),
                         preferred_element_type=jnp.float32,
                         precision=lax.Precision.HIGHEST)
    for h in range(H):
        xs_ref[h] = xs[:, h * C:(h + 1) * C]
    vs_cols = []
    vd_cols = []
    for h in range(H):
        sl = slice(h * C, (h + 1) * C)
        vs_cols.append(jnp.sum(ws_ref[:, sl] * ats_ref[:, sl], axis=1,
                               keepdims=True))
        vd_cols.append(jnp.sum(wd_ref[:, sl] * atd_ref[:, sl], axis=1,
                               keepdims=True))
    vs = jnp.concatenate(vs_cols, axis=1)
    vd = jnp.concatenate(vd_cols, axis=1)
    as_ref[...] = lax.dot_general(xb, vs, (((1,), (0,)), ((), ())),
                                  preferred_element_type=jnp.float32,
                                  precision=lax.Precision.HIGHEST)
    ad_ref[...] = lax.dot_general(xb, vd, (((1,), (0,)), ((), ())),
                                  preferred_element_type=jnp.float32,
                                  precision=lax.Precision.HIGHEST)


def _k1(x, w_src, w_dst, att_s2, att_d2):
    return pl.pallas_call(
        _k1_body,
        grid=(N // _BN1,),
        in_specs=[
            pl.BlockSpec((_BN1, D), lambda i: (i, 0)),
            pl.BlockSpec((D, HC), lambda i: (0, 0)),
            pl.BlockSpec((D, HC), lambda i: (0, 0)),
            pl.BlockSpec((1, HC), lambda i: (0, 0)),
            pl.BlockSpec((1, HC), lambda i: (0, 0)),
        ],
        out_specs=[
            pl.BlockSpec((H, _BN1, C), lambda i: (0, i, 0)),
            pl.BlockSpec((_BN1, H), lambda i: (i, 0)),
            pl.BlockSpec((_BN1, H), lambda i: (i, 0)),
        ],
        out_shape=[
            jax.ShapeDtypeStruct((H, N, C), jnp.float32),
            jax.ShapeDtypeStruct((N, H), jnp.float32),
            jax.ShapeDtypeStruct((N, H), jnp.float32),
        ],
    )(x, w_src, w_dst, att_s2, att_d2)


def _k2_body(src_hbm, dst_hbm, as_hbm, ad_hbm, ex_hbm, esum_hbm,
             src2d, dst2d, as_ts, ad_ts, ex_ts, eidx_ts, zbuf, esum_sh,
             sem):
    c = lax.axis_index("c")
    s = lax.axis_index("s")
    w = c * NS + s

    def zloop(i, _):
        zbuf[pl.ds(i * 16, 16)] = jnp.zeros((16,), jnp.float32)
        return 0
    lax.fori_loop(0, 160, zloop, 0)
    pltpu.sync_copy(zbuf, esum_sh.at[pl.ds(s * 2560, 2560)])

    pltpu.sync_copy(src_hbm.at[pl.ds(w * ROWS_W, ROWS_W)], src2d)
    pltpu.sync_copy(dst_hbm.at[pl.ds(w * ROWS_W, ROWS_W)], dst2d)
    plsc.subcore_barrier()

    def hloop(h, _):
        pltpu.sync_copy(as_hbm.at[h], as_ts)
        pltpu.sync_copy(ad_hbm.at[h], ad_ts)
        base = w * EPW

        def eloop(i, _):
            r = i // 8
            col = (i % 8) * 16
            sv = src2d[r, pl.ds(col, 16)]
            dv = dst2d[r, pl.ds(col, 16)]
            av = plsc.load_gather(as_ts, [sv]) + plsc.load_gather(ad_ts, [dv])
            av = jnp.maximum(av, 0.2 * av)
            exv = jnp.exp(av)
            gid = base + i * 16 + lax.iota(jnp.int32, 16)
            exv = jnp.where(gid < E, exv, 0.0)
            ex_ts[r, pl.ds(col, 16)] = exv
            eidx_ts[r, pl.ds(col, 16)] = dv + h * NP
            return 0
        lax.fori_loop(0, EPW // 16, eloop, 0)

        pltpu.sync_copy(ex_ts, ex_hbm.at[h, pl.ds(w * ROWS_W, ROWS_W)])
        for g in range(ROWS_W // 8):
            descs = [
                pltpu.async_copy(ex_ts.at[g * 8 + k],
                                 esum_sh.at[eidx_ts.at[g * 8 + k]],
                                 sem, add=True)
                for k in range(8)
            ]
            for d_ in descs:
                d_.wait()
        return 0
    lax.fori_loop(0, H, hloop, 0)

    plsc.subcore_barrier()
    pltpu.sync_copy(esum_sh.at[pl.ds(s * 2560, 2560)],
                    esum_hbm.at[c, pl.ds(s * 2560, 2560)])


def _k2(src2, dst2, a_s_t, a_d_t):
    return pl.kernel(
        _k2_body,
        out_type=[
            jax.ShapeDtypeStruct((H, EP // 128, 128), jnp.float32),
            jax.ShapeDtypeStruct((NC, H * NP), jnp.float32),
        ],
        mesh=_mesh(),
        compiler_params=pltpu.CompilerParams(needs_layout_passes=False),
        scratch_types=[
            pltpu.VMEM((ROWS_W, 128), jnp.int32),
            pltpu.VMEM((ROWS_W, 128), jnp.int32),
            pltpu.VMEM((N,), jnp.float32),
            pltpu.VMEM((N,), jnp.float32),
            pltpu.VMEM((ROWS_W, 128), jnp.float32),
            pltpu.VMEM((ROWS_W, 128), jnp.int32),
            pltpu.VMEM((2560,), jnp.float32),
            pltpu.VMEM_SHARED((H * NP,), jnp.float32),
            pltpu.SemaphoreType.DMA,
        ],
    )(src2, dst2, a_s_t, a_d_t)


def _k2b_body(ep_ref, o_ref):
    o_ref[...] = ep_ref[0] + ep_ref[1]


def _k2b(esum_part):
    ep3 = esum_part.reshape(NC, (H * NP) // 128, 128)
    out = pl.pallas_call(
        _k2b_body,
        out_shape=jax.ShapeDtypeStruct(((H * NP) // 128, 128), jnp.float32),
    )(ep3)
    return out.reshape(H * NP)


def _bcast16(vec, j):
    idx = jnp.full((16, 1), j, jnp.int32)
    return lax.gather(
        vec, idx,
        lax.GatherDimensionNumbers(offset_dims=(), collapsed_slice_dims=(0,),
                                   start_index_map=(0,)),
        (1,), mode=lax.GatherScatterMode.PROMISE_IN_BOUNDS)


def _k3_body(src_hbm, dst_hbm, xs_hbm, ex_hbm, esum_hbm, agg_hbm,
             gidx2d, dst2d, es_ts, wv_ts, rbuf, acc_sh, gsem, ssem):
    c = lax.axis_index("c")
    s = lax.axis_index("s")
    w = c * NS + s

    pltpu.sync_copy(src_hbm.at[pl.ds(w * ROWS_W, ROWS_W)], gidx2d)
    pltpu.sync_copy(dst_hbm.at[pl.ds(w * ROWS_W, ROWS_W)], dst2d)

    def hloop(h, _):
        pltpu.sync_copy(esum_hbm.at[pl.ds(h * NP, NP)], es_ts)
        pltpu.sync_copy(ex_hbm.at[h, pl.ds(w * ROWS_W, ROWS_W)], wv_ts)

        def wloop(i, _):
            r = i // 8
            col = (i % 8) * 16
            sl = pl.ds(col, 16)
            dv = dst2d[r, sl]
            esv = plsc.load_gather(es_ts, [dv])
            wv_ts[r, sl] = wv_ts[r, sl] / (esv + 1e-16)
            gidx2d[r, sl] = gidx2d[r, sl] + (jnp.int32(N) * (h > 0).astype(jnp.int32))
            return 0
        lax.fori_loop(0, EPW // 16, wloop, 0)

        def zloop(i, _):
            rbuf[i // 8, pl.ds((i % 8) * 16, 16)] = jnp.zeros((16,),
                                                             jnp.float32)
            return 0
        lax.fori_loop(0, 128 * 8, zloop, 0)
        for k in range(5):
            pltpu.sync_copy(rbuf, acc_sh.at[pl.ds(s * 640 + k * 128, 128)])
        plsc.subcore_barrier()

        def win(wi, _):
            pltpu.async_copy(xs_hbm.at[gidx2d.at[wi]], rbuf, gsem).wait()

            def gloop(g, _):
                w16 = wv_ts[wi, pl.ds(g * 16, 16)]

                def jloop(j, _):
                    wb = _bcast16(w16, j)
                    row = g * 16 + j
                    for k in range(8):
                        ksl = pl.ds(k * 16, 16)
                        rbuf[row, ksl] = rbuf[row, ksl] * wb
                    return 0
                lax.fori_loop(0, 16, jloop, 0)
                return 0
            lax.fori_loop(0, 8, gloop, 0)

            pltpu.async_copy(rbuf, acc_sh.at[dst2d.at[wi]], ssem,
                             add=True).wait()
            return 0
        lax.fori_loop(0, ROWS_W, win, 0)

        plsc.subcore_barrier()
        for k in range(5):
            pltpu.sync_copy(acc_sh.at[pl.ds(s * 640 + k * 128, 128)],
                            agg_hbm.at[c, h, pl.ds(s * 640 + k * 128, 128)])
        return 0

    lax.fori_loop(0, H, hloop, 0)


def _k3(src2, dst2, xs_flat, ex_buf, esum_tot):
    return pl.kernel(
        _k3_body,
        out_type=[jax.ShapeDtypeStruct((NC, H, NP, C), jnp.float32)],
        mesh=_mesh(),
        compiler_params=pltpu.CompilerParams(needs_layout_passes=False),
        scratch_types=[
            pltpu.VMEM((ROWS_W, 128), jnp.int32),
            pltpu.VMEM((ROWS_W, 128), jnp.int32),
            pltpu.VMEM((NP,), jnp.float32),
            pltpu.VMEM((ROWS_W, 128), jnp.float32),
            pltpu.VMEM((128, C), jnp.float32),
            pltpu.VMEM_SHARED((NP, C), jnp.float32),
            pltpu.SemaphoreType.DMA,
            pltpu.SemaphoreType.DMA,
        ],
    )(src2, dst2, xs_flat, ex_buf, esum_tot)


_BN4 = 2000

def _k4_body(agg_ref, b_ref, g_ref, be_ref, pw_ref, o_ref):
    sh = []
    tot = jnp.zeros((_BN4, 1), jnp.float32)
    for h in range(H):
        sl = slice(h * C, (h + 1) * C)
        v = agg_ref[0, h] + agg_ref[1, h] + b_ref[:, sl]
        sh.append(v)
        tot = tot + jnp.sum(v, axis=1, keepdims=True)
    mu = tot / HC
    var = jnp.zeros((_BN4, 1), jnp.float32)
    for h in range(H):
        d = sh[h] - mu
        var = var + jnp.sum(d * d, axis=1, keepdims=True)
    inv = 1.0 / jnp.sqrt(var / HC + 1e-5)
    for h in range(H):
        sl = slice(h * C, (h + 1) * C)
        y = (sh[h] - mu) * inv * g_ref[:, sl] + be_ref[:, sl]
        o_ref[:, sl] = jnp.where(y > 0, y, pw_ref[:, sl] * y)


def _k4(agg, b2, g2, be2, pw2):
    return pl.pallas_call(
        _k4_body,
        grid=(N // _BN4,),
        in_specs=[
            pl.BlockSpec((NC, H, _BN4, C), lambda i: (0, 0, i, 0)),
            pl.BlockSpec((1, HC), lambda i: (0, 0)),
            pl.BlockSpec((1, HC), lambda i: (0, 0)),
            pl.BlockSpec((1, HC), lambda i: (0, 0)),
            pl.BlockSpec((1, HC), lambda i: (0, 0)),
        ],
        out_specs=pl.BlockSpec((_BN4, HC), lambda i: (i, 0)),
        out_shape=jax.ShapeDtypeStruct((N, HC), jnp.float32),
    )(agg, b2, g2, be2, pw2)


def kernel(x, edge_attr, edge_index, batch, W_src, W_dst, att_src, att_dst,
           bias, ln_gamma, ln_beta, prelu_w):
    src = edge_index[0]
    dst = edge_index[1]
    pad = jnp.zeros((EP - E,), jnp.int32)
    src2 = jnp.concatenate([src, pad]).reshape(EP // 128, 128)
    dst2 = jnp.concatenate([dst, pad]).reshape(EP // 128, 128)

    att_s2 = att_src.reshape(1, HC)
    att_d2 = att_dst.reshape(1, HC)

    xs3, a_src, a_dst = _k1(x, W_src, W_dst, att_s2, att_d2)
    a_s_t = a_src.T.reshape(H, N)
    a_d_t = a_dst.T.reshape(H, N)
    xs_flat = xs3.reshape(H * N, C)

    ex_buf, esum_part = _k2(src2, dst2, a_s_t, a_d_t)
    esum_tot = _k2b(esum_part)
    (agg,) = _k3(src2, dst2, xs_flat, ex_buf, esum_tot)

    b2 = bias.reshape(1, HC)
    g2 = ln_gamma.reshape(1, HC)
    be2 = ln_beta.reshape(1, HC)
    pw2 = prelu_w.reshape(1, HC)
    return _k4(agg, b2, g2, be2, pw2)


# speedup vs baseline: 15.9969x; 1.0915x over previous
import functools

import jax
import jax.numpy as jnp
from jax import lax
from jax.experimental import pallas as pl
from jax.experimental.pallas import tpu as pltpu
from jax.experimental.pallas import tpu_sc as plsc

N = 10000
E = 160000
D = 256
C = 128
H = 4
HC = H * C

NC = 2
NS = 16
NW = NC * NS
NP = 10240
EP = 163840
EPW = EP // NW
WN = 128
NWIN = EPW // WN
NPAIR = NWIN // 2


@functools.cache
def _mesh():
    return plsc.VectorSubcoreMesh(
        core_axis_name="c", subcore_axis_name="s",
        num_cores=NC, num_subcores=NS)


_BN1 = 2000

def _k1_body(x_ref, ws_ref, wd_ref, ats_ref, atd_ref, xs_ref, as_ref, ad_ref):
    xb = x_ref[...]
    xs = lax.dot_general(xb, ws_ref[...], (((1,), (0,)), ((), ())),
                         preferred_element_type=jnp.float32,
                         precision=lax.Precision.HIGHEST)
    for h in range(H):
        xs_ref[h] = xs[:, h * C:(h + 1) * C]
    vs_cols = []
    vd_cols = []
    for h in range(H):
        sl = slice(h * C, (h + 1) * C)
        vs_cols.append(jnp.sum(ws_ref[:, sl] * ats_ref[:, sl], axis=1,
                               keepdims=True))
        vd_cols.append(jnp.sum(wd_ref[:, sl] * atd_ref[:, sl], axis=1,
                               keepdims=True))
    vs = jnp.concatenate(vs_cols, axis=1)
    vd = jnp.concatenate(vd_cols, axis=1)
    as_ref[...] = lax.dot_general(xb, vs, (((1,), (0,)), ((), ())),
                                  preferred_element_type=jnp.float32,
                                  precision=lax.Precision.HIGHEST)
    ad_ref[...] = lax.dot_general(xb, vd, (((1,), (0,)), ((), ())),
                                  preferred_element_type=jnp.float32,
                                  precision=lax.Precision.HIGHEST)


def _k1(x, w_src, w_dst, att_s2, att_d2):
    return pl.pallas_call(
        _k1_body,
        grid=(N // _BN1,),
        in_specs=[
            pl.BlockSpec((_BN1, D), lambda i: (i, 0)),
            pl.BlockSpec((D, HC), lambda i: (0, 0)),
            pl.BlockSpec((D, HC), lambda i: (0, 0)),
            pl.BlockSpec((1, HC), lambda i: (0, 0)),
            pl.BlockSpec((1, HC), lambda i: (0, 0)),
        ],
        out_specs=[
            pl.BlockSpec((H, _BN1, C), lambda i: (0, i, 0)),
            pl.BlockSpec((_BN1, H), lambda i: (i, 0)),
            pl.BlockSpec((_BN1, H), lambda i: (i, 0)),
        ],
        out_shape=[
            jax.ShapeDtypeStruct((H, N, C), jnp.float32),
            jax.ShapeDtypeStruct((N, H), jnp.float32),
            jax.ShapeDtypeStruct((N, H), jnp.float32),
        ],
    )(x, w_src, w_dst, att_s2, att_d2)


def _k2_body(src_hbm, dst_hbm, as_hbm, ad_hbm, ex_hbm, esum_hbm,
             src2d, dst2d, as_ts, ad_ts, ex_ts, eidx_ts, zbuf, esum_sh,
             sem):
    c = lax.axis_index("c")
    s = lax.axis_index("s")
    w = c * NS + s

    def zloop(i, _):
        zbuf[pl.ds(i * 16, 16)] = jnp.zeros((16,), jnp.float32)
        return 0
    lax.fori_loop(0, 160, zloop, 0)
    pltpu.sync_copy(zbuf, esum_sh.at[pl.ds(s * 2560, 2560)])

    pltpu.sync_copy(src_hbm.at[pl.ds(w * NWIN, NWIN)], src2d)
    pltpu.sync_copy(dst_hbm.at[pl.ds(w * NWIN, NWIN)], dst2d)
    plsc.subcore_barrier()

    def hloop(h, _):
        pltpu.sync_copy(as_hbm.at[h], as_ts)
        pltpu.sync_copy(ad_hbm.at[h], ad_ts)
        base = w * EPW

        def eloop(i, _):
            r = i // 8
            col = (i % 8) * 16
            sv = src2d[r, pl.ds(col, 16)]
            dv = dst2d[r, pl.ds(col, 16)]
            av = plsc.load_gather(as_ts, [sv]) + plsc.load_gather(ad_ts, [dv])
            av = jnp.maximum(av, 0.2 * av)
            exv = jnp.exp(av)
            gid = base + i * 16 + lax.iota(jnp.int32, 16)
            exv = jnp.where(gid < E, exv, 0.0)
            ex_ts[r, pl.ds(col, 16)] = exv
            eidx_ts[r, pl.ds(col, 16)] = dv + h * NP
            return 0
        lax.fori_loop(0, EPW // 16, eloop, 0)

        pltpu.sync_copy(ex_ts, ex_hbm.at[h, pl.ds(w * NWIN, NWIN)])
        for g in range(NWIN // 8):
            descs = [
                pltpu.async_copy(ex_ts.at[g * 8 + k],
                                 esum_sh.at[eidx_ts.at[g * 8 + k]],
                                 sem, add=True)
                for k in range(8)
            ]
            for d_ in descs:
                d_.wait()
        return 0
    lax.fori_loop(0, H, hloop, 0)

    plsc.subcore_barrier()
    pltpu.sync_copy(esum_sh.at[pl.ds(s * 2560, 2560)],
                    esum_hbm.at[c, pl.ds(s * 2560, 2560)])


def _k2(src2, dst2, a_s_t, a_d_t):
    return pl.kernel(
        _k2_body,
        out_type=[
            jax.ShapeDtypeStruct((H, EP // WN, WN), jnp.float32),
            jax.ShapeDtypeStruct((NC, H * NP), jnp.float32),
        ],
        mesh=_mesh(),
        compiler_params=pltpu.CompilerParams(needs_layout_passes=False),
        scratch_types=[
            pltpu.VMEM((NWIN, WN), jnp.int32),
            pltpu.VMEM((NWIN, WN), jnp.int32),
            pltpu.VMEM((N,), jnp.float32),
            pltpu.VMEM((N,), jnp.float32),
            pltpu.VMEM((NWIN, WN), jnp.float32),
            pltpu.VMEM((NWIN, WN), jnp.int32),
            pltpu.VMEM((2560,), jnp.float32),
            pltpu.VMEM_SHARED((H * NP,), jnp.float32),
            pltpu.SemaphoreType.DMA,
        ],
    )(src2, dst2, a_s_t, a_d_t)


def _k2b_body(ep_ref, o_ref):
    o_ref[...] = ep_ref[0] + ep_ref[1]


def _k2b(esum_part):
    ep3 = esum_part.reshape(NC, (H * NP) // 128, 128)
    out = pl.pallas_call(
        _k2b_body,
        out_shape=jax.ShapeDtypeStruct(((H * NP) // 128, 128), jnp.float32),
    )(ep3)
    return out


def _bcast16(vec, j):
    idx = jnp.full((16, 1), j, jnp.int32)
    return lax.gather(
        vec, idx,
        lax.GatherDimensionNumbers(offset_dims=(), collapsed_slice_dims=(0,),
                                   start_index_map=(0,)),
        (1,), mode=lax.GatherScatterMode.PROMISE_IN_BOUNDS)


def _k3_body(src_hbm, dst_hbm, xs_hbm, ex_hbm, esum_hbm, agg_hbm,
             gidx2d, dst2d, wv_ts, rbuf0, rbuf1, acc_sh,
             gsem0, gsem1, ssem0, ssem1):
    c = lax.axis_index("c")
    s = lax.axis_index("s")
    w = c * NS + s

    pltpu.sync_copy(src_hbm.at[pl.ds(w * NWIN, NWIN)], gidx2d)
    pltpu.sync_copy(dst_hbm.at[pl.ds(w * NWIN, NWIN)], dst2d)

    def scale(buf, wi):
        def gloop(g, _):
            w16 = wv_ts[wi, pl.ds(g * 16, 16)]
            for j in range(16):
                wb = _bcast16(w16, j)
                row = g * 16 + j
                for k in range(8):
                    ksl = pl.ds(k * 16, 16)
                    buf[row, ksl] = buf[row, ksl] * wb
            return 0
        lax.fori_loop(0, WN // 16, gloop, 0)

    def hloop(h, _):
        pltpu.sync_copy(esum_hbm.at[pl.ds(h * (NP // 128), NP // 128)],
                        rbuf0.at[pl.ds(0, NP // 128)])
        pltpu.sync_copy(ex_hbm.at[h, pl.ds(w * NWIN, NWIN)], wv_ts)

        def wloop(i, _):
            r = i // 8
            col = (i % 8) * 16
            sl = pl.ds(col, 16)
            dv = dst2d[r, sl]
            esv = plsc.load_gather(rbuf0, [dv >> 7, dv & 127])
            wv_ts[r, sl] = wv_ts[r, sl] / (esv + 1e-16)
            gidx2d[r, sl] = gidx2d[r, sl] + (
                jnp.int32(N) * (h > 0).astype(jnp.int32))
            return 0
        lax.fori_loop(0, EPW // 16, wloop, 0)

        def zloop(i, _):
            rbuf0[i // 8, pl.ds((i % 8) * 16, 16)] = jnp.zeros((16,),
                                                              jnp.float32)
            return 0
        lax.fori_loop(0, WN * 8, zloop, 0)
        for k in range(5):
            pltpu.sync_copy(rbuf0, acc_sh.at[pl.ds(s * 640 + k * WN, WN)])
        plsc.subcore_barrier()

        pltpu.async_copy(xs_hbm.at[gidx2d.at[0]], rbuf0, gsem0)

        def pair(pi, _):
            w0 = 2 * pi
            w1 = w0 + 1
            pltpu.make_async_copy(xs_hbm.at[gidx2d.at[w0]], rbuf0,
                                  gsem0).wait()
            d1 = pltpu.async_copy(xs_hbm.at[gidx2d.at[w1]], rbuf1, gsem1)
            scale(rbuf0, w0)
            s0 = pltpu.async_copy(rbuf0, acc_sh.at[dst2d.at[w0]], ssem0,
                                  add=True)
            d1.wait()
            scale(rbuf1, w1)
            s0.wait()
            nxt = jnp.where(pi < NPAIR - 1, w0 + 2, 0)
            pltpu.async_copy(xs_hbm.at[gidx2d.at[nxt]], rbuf0, gsem0)
            s1 = pltpu.async_copy(rbuf1, acc_sh.at[dst2d.at[w1]], ssem1,
                                  add=True)
            s1.wait()
            return 0
        lax.fori_loop(0, NPAIR, pair, 0)
        pltpu.make_async_copy(xs_hbm.at[gidx2d.at[0]], rbuf0, gsem0).wait()

        plsc.subcore_barrier()
        for k in range(5):
            pltpu.sync_copy(acc_sh.at[pl.ds(s * 640 + k * WN, WN)],
                            agg_hbm.at[c, h, pl.ds(s * 640 + k * WN, WN)])
        return 0

    lax.fori_loop(0, H, hloop, 0)


def _k3(src2, dst2, xs_flat, ex_buf, esum_tot):
    return pl.kernel(
        _k3_body,
        out_type=[jax.ShapeDtypeStruct((NC, H, NP, C), jnp.float32)],
        mesh=_mesh(),
        compiler_params=pltpu.CompilerParams(needs_layout_passes=False),
        scratch_types=[
            pltpu.VMEM((NWIN, WN), jnp.int32),
            pltpu.VMEM((NWIN, WN), jnp.int32),
            pltpu.VMEM((NWIN, WN), jnp.float32),
            pltpu.VMEM((WN, C), jnp.float32),
            pltpu.VMEM((WN, C), jnp.float32),
            pltpu.VMEM_SHARED((NP, C), jnp.float32),
            pltpu.SemaphoreType.DMA,
            pltpu.SemaphoreType.DMA,
            pltpu.SemaphoreType.DMA,
            pltpu.SemaphoreType.DMA,
        ],
    )(src2, dst2, xs_flat, ex_buf, esum_tot)


_BN4 = 2000

def _k4_body(agg_ref, b_ref, g_ref, be_ref, pw_ref, o_ref):
    sh = []
    tot = jnp.zeros((_BN4, 1), jnp.float32)
    for h in range(H):
        sl = slice(h * C, (h + 1) * C)
        v = agg_ref[0, h] + agg_ref[1, h] + b_ref[:, sl]
        sh.append(v)
        tot = tot + jnp.sum(v, axis=1, keepdims=True)
    mu = tot / HC
    var = jnp.zeros((_BN4, 1), jnp.float32)
    for h in range(H):
        d = sh[h] - mu
        var = var + jnp.sum(d * d, axis=1, keepdims=True)
    inv = 1.0 / jnp.sqrt(var / HC + 1e-5)
    for h in range(H):
        sl = slice(h * C, (h + 1) * C)
        y = (sh[h] - mu) * inv * g_ref[:, sl] + be_ref[:, sl]
        o_ref[:, sl] = jnp.where(y > 0, y, pw_ref[:, sl] * y)


def _k4(agg, b2, g2, be2, pw2):
    return pl.pallas_call(
        _k4_body,
        grid=(N // _BN4,),
        in_specs=[
            pl.BlockSpec((NC, H, _BN4, C), lambda i: (0, 0, i, 0)),
            pl.BlockSpec((1, HC), lambda i: (0, 0)),
            pl.BlockSpec((1, HC), lambda i: (0, 0)),
            pl.BlockSpec((1, HC), lambda i: (0, 0)),
            pl.BlockSpec((1, HC), lambda i: (0, 0)),
        ],
        out_specs=pl.BlockSpec((_BN4, HC), lambda i: (i, 0)),
        out_shape=jax.ShapeDtypeStruct((N, HC), jnp.float32),
    )(agg, b2, g2, be2, pw2)


def kernel(x, edge_attr, edge_index, batch, W_src, W_dst, att_src, att_dst,
           bias, ln_gamma, ln_beta, prelu_w):
    src = edge_index[0]
    dst = edge_index[1]
    pad = jnp.zeros((EP - E,), jnp.int32)
    src2 = jnp.concatenate([src, pad]).reshape(EP // WN, WN)
    dst2 = jnp.concatenate([dst, pad]).reshape(EP // WN, WN)

    att_s2 = att_src.reshape(1, HC)
    att_d2 = att_dst.reshape(1, HC)

    xs3, a_src, a_dst = _k1(x, W_src, W_dst, att_s2, att_d2)
    a_s_t = a_src.T.reshape(H, N)
    a_d_t = a_dst.T.reshape(H, N)
    xs_flat = xs3.reshape(H * N, C)

    ex_buf, esum_part = _k2(src2, dst2, a_s_t, a_d_t)
    esum_tot = _k2b(esum_part)
    (agg,) = _k3(src2, dst2, xs_flat, ex_buf, esum_tot)

    b2 = bias.reshape(1, HC)
    g2 = ln_gamma.reshape(1, HC)
    be2 = ln_beta.reshape(1, HC)
    pw2 = prelu_w.reshape(1, HC)
    return _k4(agg, b2, g2, be2, pw2)


# speedup vs baseline: 38.0607x; 2.3793x over previous
import functools

import jax
import jax.numpy as jnp
from jax import lax
from jax.experimental import pallas as pl
from jax.experimental.pallas import tpu as pltpu
from jax.experimental.pallas import tpu_sc as plsc

N = 10000
E = 160000
D = 256
C = 128
H = 4
HC = H * C

NC = 2
NS = 16
NW = NC * NS
NP = 10240
EP = 163840
EPW = EP // NW
WN = 128
NWIN = EPW // WN
NPAIR = NWIN // 2


@functools.cache
def _mesh():
    return plsc.VectorSubcoreMesh(
        core_axis_name="c", subcore_axis_name="s",
        num_cores=NC, num_subcores=NS)


_BN1 = 2000

def _k1_body(x_ref, ws_ref, wd_ref, ats_ref, atd_ref, xs_ref, as_ref, ad_ref):
    xb = x_ref[...]
    xs = lax.dot_general(xb, ws_ref[...], (((1,), (0,)), ((), ())),
                         preferred_element_type=jnp.float32,
                         precision=lax.Precision.HIGHEST)
    for h in range(H):
        xs_ref[h] = xs[:, h * C:(h + 1) * C]
    vs_cols = []
    vd_cols = []
    for h in range(H):
        sl = slice(h * C, (h + 1) * C)
        vs_cols.append(jnp.sum(ws_ref[:, sl] * ats_ref[:, sl], axis=1,
                               keepdims=True))
        vd_cols.append(jnp.sum(wd_ref[:, sl] * atd_ref[:, sl], axis=1,
                               keepdims=True))
    vs = jnp.concatenate(vs_cols, axis=1)
    vd = jnp.concatenate(vd_cols, axis=1)
    as_ref[...] = lax.dot_general(xb, vs, (((1,), (0,)), ((), ())),
                                  preferred_element_type=jnp.float32,
                                  precision=lax.Precision.HIGHEST)
    ad_ref[...] = lax.dot_general(xb, vd, (((1,), (0,)), ((), ())),
                                  preferred_element_type=jnp.float32,
                                  precision=lax.Precision.HIGHEST)


def _k1(x, w_src, w_dst, att_s2, att_d2):
    return pl.pallas_call(
        _k1_body,
        grid=(N // _BN1,),
        in_specs=[
            pl.BlockSpec((_BN1, D), lambda i: (i, 0)),
            pl.BlockSpec((D, HC), lambda i: (0, 0)),
            pl.BlockSpec((D, HC), lambda i: (0, 0)),
            pl.BlockSpec((1, HC), lambda i: (0, 0)),
            pl.BlockSpec((1, HC), lambda i: (0, 0)),
        ],
        out_specs=[
            pl.BlockSpec((H, _BN1, C), lambda i: (0, i, 0)),
            pl.BlockSpec((_BN1, H), lambda i: (i, 0)),
            pl.BlockSpec((_BN1, H), lambda i: (i, 0)),
        ],
        out_shape=[
            jax.ShapeDtypeStruct((H, N, C), jnp.float32),
            jax.ShapeDtypeStruct((N, H), jnp.float32),
            jax.ShapeDtypeStruct((N, H), jnp.float32),
        ],
    )(x, w_src, w_dst, att_s2, att_d2)


def _k2_body(src_hbm, dst_hbm, as_hbm, ad_hbm, ex_hbm, esum_hbm,
             src2d, dst2d, as_ts, ad_ts, ex_ts, eidx_ts, zbuf, esum_sh,
             sem):
    c = lax.axis_index("c")
    s = lax.axis_index("s")
    w = c * NS + s

    def zloop(i, _):
        zbuf[pl.ds(i * 16, 16)] = jnp.zeros((16,), jnp.float32)
        return 0
    lax.fori_loop(0, 160, zloop, 0)
    pltpu.sync_copy(zbuf, esum_sh.at[pl.ds(s * 2560, 2560)])

    pltpu.sync_copy(src_hbm.at[pl.ds(w * NWIN, NWIN)], src2d)
    pltpu.sync_copy(dst_hbm.at[pl.ds(w * NWIN, NWIN)], dst2d)
    plsc.subcore_barrier()

    def hloop(h, _):
        pltpu.sync_copy(as_hbm.at[h], as_ts)
        pltpu.sync_copy(ad_hbm.at[h], ad_ts)
        base = w * EPW

        def eloop(i, _):
            r = i // 8
            col = (i % 8) * 16
            sv = src2d[r, pl.ds(col, 16)]
            dv = dst2d[r, pl.ds(col, 16)]
            av = plsc.load_gather(as_ts, [sv]) + plsc.load_gather(ad_ts, [dv])
            av = jnp.maximum(av, 0.2 * av)
            exv = jnp.exp(av)
            gid = base + i * 16 + lax.iota(jnp.int32, 16)
            exv = jnp.where(gid < E, exv, 0.0)
            ex_ts[r, pl.ds(col, 16)] = exv
            eidx_ts[r, pl.ds(col, 16)] = dv + h * NP
            return 0
        lax.fori_loop(0, EPW // 16, eloop, 0)

        pltpu.sync_copy(ex_ts, ex_hbm.at[h, pl.ds(w * NWIN, NWIN)])
        for g in range(NWIN // 8):
            descs = [
                pltpu.async_copy(ex_ts.at[g * 8 + k],
                                 esum_sh.at[eidx_ts.at[g * 8 + k]],
                                 sem, add=True)
                for k in range(8)
            ]
            for d_ in descs:
                d_.wait()
        return 0
    lax.fori_loop(0, H, hloop, 0)

    plsc.subcore_barrier()
    pltpu.sync_copy(esum_sh.at[pl.ds(s * 2560, 2560)],
                    esum_hbm.at[c, pl.ds(s * 2560, 2560)])


def _k2(src2, dst2, a_s_t, a_d_t):
    return pl.kernel(
        _k2_body,
        out_type=[
            jax.ShapeDtypeStruct((H, EP // WN, WN), jnp.float32),
            jax.ShapeDtypeStruct((NC, H * NP), jnp.float32),
        ],
        mesh=_mesh(),
        compiler_params=pltpu.CompilerParams(needs_layout_passes=False),
        scratch_types=[
            pltpu.VMEM((NWIN, WN), jnp.int32),
            pltpu.VMEM((NWIN, WN), jnp.int32),
            pltpu.VMEM((N,), jnp.float32),
            pltpu.VMEM((N,), jnp.float32),
            pltpu.VMEM((NWIN, WN), jnp.float32),
            pltpu.VMEM((NWIN, WN), jnp.int32),
            pltpu.VMEM((2560,), jnp.float32),
            pltpu.VMEM_SHARED((H * NP,), jnp.float32),
            pltpu.SemaphoreType.DMA,
        ],
    )(src2, dst2, a_s_t, a_d_t)


def _k2b_body(ep_ref, o_ref):
    o_ref[...] = ep_ref[0] + ep_ref[1]


def _k2b(esum_part):
    ep3 = esum_part.reshape(NC, (H * NP) // 128, 128)
    out = pl.pallas_call(
        _k2b_body,
        out_shape=jax.ShapeDtypeStruct(((H * NP) // 128, 128), jnp.float32),
    )(ep3)
    return out


def _bcast16(vec, j):
    idx = jnp.full((16, 1), j, jnp.int32)
    return lax.gather(
        vec, idx,
        lax.GatherDimensionNumbers(offset_dims=(), collapsed_slice_dims=(0,),
                                   start_index_map=(0,)),
        (1,), mode=lax.GatherScatterMode.PROMISE_IN_BOUNDS)


def _k3_body(src_hbm, dst_hbm, xs_hbm, ex_hbm, esum_hbm, agg_hbm,
             gidx2d, dst2d, wv_ts, rbuf0, rbuf1, acc_sh,
             gsem0, gsem1, ssem0, ssem1):
    c = lax.axis_index("c")
    s = lax.axis_index("s")
    w = c * NS + s

    pltpu.sync_copy(src_hbm.at[pl.ds(w * NWIN, NWIN)], gidx2d)
    pltpu.sync_copy(dst_hbm.at[pl.ds(w * NWIN, NWIN)], dst2d)

    def scale(buf, wi):
        def gloop(g, _):
            w16 = wv_ts[wi, pl.ds(g * 16, 16)]
            for j in range(16):
                wb = _bcast16(w16, j)
                row = g * 16 + j
                for k in range(8):
                    ksl = pl.ds(k * 16, 16)
                    buf[row, ksl] = buf[row, ksl] * wb
            return 0
        lax.fori_loop(0, WN // 16, gloop, 0)

    def hloop(h, _):
        pltpu.sync_copy(esum_hbm.at[pl.ds(h * (NP // 128), NP // 128)],
                        rbuf0.at[pl.ds(0, NP // 128)])
        pltpu.sync_copy(ex_hbm.at[h, pl.ds(w * NWIN, NWIN)], wv_ts)

        def wloop(i, _):
            r = i // 8
            col = (i % 8) * 16
            sl = pl.ds(col, 16)
            dv = dst2d[r, sl]
            esv = plsc.load_gather(rbuf0, [dv >> 7, dv & 127])
            wv_ts[r, sl] = wv_ts[r, sl] / (esv + 1e-16)
            gidx2d[r, sl] = gidx2d[r, sl] + (
                jnp.int32(N) * (h > 0).astype(jnp.int32))
            return 0
        lax.fori_loop(0, EPW // 16, wloop, 0)

        def zloop(i, _):
            rbuf0[i // 8, pl.ds((i % 8) * 16, 16)] = jnp.zeros((16,),
                                                              jnp.float32)
            return 0
        lax.fori_loop(0, WN * 8, zloop, 0)
        for k in range(5):
            pltpu.sync_copy(rbuf0, acc_sh.at[pl.ds(s * 640 + k * WN, WN)])
        plsc.subcore_barrier()

        pltpu.async_copy(xs_hbm.at[gidx2d.at[0]], rbuf0, gsem0)

        def pair(pi, _):
            w0 = 2 * pi
            w1 = w0 + 1
            pltpu.make_async_copy(xs_hbm.at[gidx2d.at[w0]], rbuf0,
                                  gsem0).wait()
            d1 = pltpu.async_copy(xs_hbm.at[gidx2d.at[w1]], rbuf1, gsem1)
            scale(rbuf0, w0)
            s0 = pltpu.async_copy(rbuf0, acc_sh.at[dst2d.at[w0]], ssem0,
                                  add=True)
            d1.wait()
            scale(rbuf1, w1)
            s0.wait()
            nxt = jnp.where(pi < NPAIR - 1, w0 + 2, 0)
            pltpu.async_copy(xs_hbm.at[gidx2d.at[nxt]], rbuf0, gsem0)
            s1 = pltpu.async_copy(rbuf1, acc_sh.at[dst2d.at[w1]], ssem1,
                                  add=True)
            s1.wait()
            return 0
        lax.fori_loop(0, NPAIR, pair, 0)
        pltpu.make_async_copy(xs_hbm.at[gidx2d.at[0]], rbuf0, gsem0).wait()

        plsc.subcore_barrier()
        for k in range(5):
            pltpu.sync_copy(acc_sh.at[pl.ds(s * 640 + k * WN, WN)],
                            agg_hbm.at[c, h, pl.ds(s * 640 + k * WN, WN)])
        return 0

    lax.fori_loop(0, H, hloop, 0)


def _k3(src2, dst2, xs_flat, ex_buf, esum_tot):
    return pl.kernel(
        _k3_body,
        out_type=[jax.ShapeDtypeStruct((NC, H, NP, C), jnp.float32)],
        mesh=_mesh(),
        compiler_params=pltpu.CompilerParams(needs_layout_passes=False),
        scratch_types=[
            pltpu.VMEM((NWIN, WN), jnp.int32),
            pltpu.VMEM((NWIN, WN), jnp.int32),
            pltpu.VMEM((NWIN, WN), jnp.float32),
            pltpu.VMEM((WN, C), jnp.float32),
            pltpu.VMEM((WN, C), jnp.float32),
            pltpu.VMEM_SHARED((NP, C), jnp.float32),
            pltpu.SemaphoreType.DMA,
            pltpu.SemaphoreType.DMA,
            pltpu.SemaphoreType.DMA,
            pltpu.SemaphoreType.DMA,
        ],
    )(src2, dst2, xs_flat, ex_buf, esum_tot)


_BN4 = 2000

def _k4_body(agg_ref, b_ref, g_ref, be_ref, pw_ref, o_ref):
    sh = []
    tot = jnp.zeros((_BN4, 1), jnp.float32)
    for h in range(H):
        sl = slice(h * C, (h + 1) * C)
        v = agg_ref[0, h] + agg_ref[1, h] + b_ref[:, sl]
        sh.append(v)
        tot = tot + jnp.sum(v, axis=1, keepdims=True)
    mu = tot / HC
    var = jnp.zeros((_BN4, 1), jnp.float32)
    for h in range(H):
        d = sh[h] - mu
        var = var + jnp.sum(d * d, axis=1, keepdims=True)
    inv = 1.0 / jnp.sqrt(var / HC + 1e-5)
    for h in range(H):
        sl = slice(h * C, (h + 1) * C)
        y = (sh[h] - mu) * inv * g_ref[:, sl] + be_ref[:, sl]
        o_ref[:, sl] = jnp.where(y > 0, y, pw_ref[:, sl] * y)


def _k4(agg, b2, g2, be2, pw2):
    return pl.pallas_call(
        _k4_body,
        grid=(N // _BN4,),
        in_specs=[
            pl.BlockSpec((NC, H, _BN4, C), lambda i: (0, 0, i, 0)),
            pl.BlockSpec((1, HC), lambda i: (0, 0)),
            pl.BlockSpec((1, HC), lambda i: (0, 0)),
            pl.BlockSpec((1, HC), lambda i: (0, 0)),
            pl.BlockSpec((1, HC), lambda i: (0, 0)),
        ],
        out_specs=pl.BlockSpec((_BN4, HC), lambda i: (i, 0)),
        out_shape=jax.ShapeDtypeStruct((N, HC), jnp.float32),
    )(agg, b2, g2, be2, pw2)


def kernel(x, edge_attr, edge_index, batch, W_src, W_dst, att_src, att_dst,
           bias, ln_gamma, ln_beta, prelu_w):
    src = edge_index[0]
    dst = edge_index[1]
    pad = (jnp.arange(EP - E, dtype=jnp.int32) * 37) % N
    src2 = jnp.concatenate([src, pad]).reshape(EP // WN, WN)
    dst2 = jnp.concatenate([dst, pad]).reshape(EP // WN, WN)

    att_s2 = att_src.reshape(1, HC)
    att_d2 = att_dst.reshape(1, HC)

    xs3, a_src, a_dst = _k1(x, W_src, W_dst, att_s2, att_d2)
    a_s_t = a_src.T.reshape(H, N)
    a_d_t = a_dst.T.reshape(H, N)
    xs_flat = xs3.reshape(H * N, C)

    ex_buf, esum_part = _k2(src2, dst2, a_s_t, a_d_t)
    esum_tot = _k2b(esum_part)
    (agg,) = _k3(src2, dst2, xs_flat, ex_buf, esum_tot)

    b2 = bias.reshape(1, HC)
    g2 = ln_gamma.reshape(1, HC)
    be2 = ln_beta.reshape(1, HC)
    pw2 = prelu_w.reshape(1, HC)
    return _k4(agg, b2, g2, be2, pw2)


# speedup vs baseline: 39.1469x; 1.0285x over previous
import functools

import jax
import jax.numpy as jnp
from jax import lax
from jax.experimental import pallas as pl
from jax.experimental.pallas import tpu as pltpu
from jax.experimental.pallas import tpu_sc as plsc

N = 10000
E = 160000
D = 256
C = 128
H = 4
HC = H * C

NC = 2
NS = 16
NW = NC * NS
NP = 10240
EP = 163840
EPW = EP // NW
WN = 128
NWIN = EPW // WN
NPAIR = NWIN // 2


@functools.cache
def _mesh():
    return plsc.VectorSubcoreMesh(
        core_axis_name="c", subcore_axis_name="s",
        num_cores=NC, num_subcores=NS)


_BN1 = 2000

def _k1a_body(x_ref, ws_ref, wd_ref, ats_ref, atd_ref, as_ref, ad_ref):
    xb = x_ref[...]
    vs_cols = []
    vd_cols = []
    for h in range(H):
        sl = slice(h * C, (h + 1) * C)
        vs_cols.append(jnp.sum(ws_ref[:, sl] * ats_ref[:, sl], axis=1,
                               keepdims=True))
        vd_cols.append(jnp.sum(wd_ref[:, sl] * atd_ref[:, sl], axis=1,
                               keepdims=True))
    vs = jnp.concatenate(vs_cols, axis=1)
    vd = jnp.concatenate(vd_cols, axis=1)
    as_ref[...] = lax.dot_general(xb, vs, (((1,), (0,)), ((), ())),
                                  preferred_element_type=jnp.float32,
                                  precision=lax.Precision.HIGHEST)
    ad_ref[...] = lax.dot_general(xb, vd, (((1,), (0,)), ((), ())),
                                  preferred_element_type=jnp.float32,
                                  precision=lax.Precision.HIGHEST)


def _k1a(x, w_src, w_dst, att_s2, att_d2):
    return pl.pallas_call(
        _k1a_body,
        grid=(N // _BN1,),
        in_specs=[
            pl.BlockSpec((_BN1, D), lambda i: (i, 0)),
            pl.BlockSpec((D, HC), lambda i: (0, 0)),
            pl.BlockSpec((D, HC), lambda i: (0, 0)),
            pl.BlockSpec((1, HC), lambda i: (0, 0)),
            pl.BlockSpec((1, HC), lambda i: (0, 0)),
        ],
        out_specs=[
            pl.BlockSpec((_BN1, H), lambda i: (i, 0)),
            pl.BlockSpec((_BN1, H), lambda i: (i, 0)),
        ],
        out_shape=[
            jax.ShapeDtypeStruct((N, H), jnp.float32),
            jax.ShapeDtypeStruct((N, H), jnp.float32),
        ],
    )(x, w_src, w_dst, att_s2, att_d2)


def _k1b_body(x_ref, ws_ref, xs_ref):
    xs = lax.dot_general(x_ref[...], ws_ref[...], (((1,), (0,)), ((), ())),
                         preferred_element_type=jnp.float32,
                         precision=lax.Precision.HIGHEST)
    for h in range(H):
        xs_ref[h] = xs[:, h * C:(h + 1) * C]


def _k1b(x, w_src):
    return pl.pallas_call(
        _k1b_body,
        grid=(N // _BN1,),
        in_specs=[
            pl.BlockSpec((_BN1, D), lambda i: (i, 0)),
            pl.BlockSpec((D, HC), lambda i: (0, 0)),
        ],
        out_specs=pl.BlockSpec((H, _BN1, C), lambda i: (0, i, 0)),
        out_shape=jax.ShapeDtypeStruct((H, N, C), jnp.float32),
    )(x, w_src)


def _k2_body(src_hbm, dst_hbm, as_hbm, ad_hbm, ex_hbm, esum_hbm,
             src2d, dst2d, as_ts, ad_ts, ex_ts, eidx_ts, zbuf, esum_sh,
             sem):
    c = lax.axis_index("c")
    s = lax.axis_index("s")
    w = c * NS + s

    def zloop(i, _):
        zbuf[pl.ds(i * 16, 16)] = jnp.zeros((16,), jnp.float32)
        return 0
    lax.fori_loop(0, 160, zloop, 0)
    pltpu.sync_copy(zbuf, esum_sh.at[pl.ds(s * 2560, 2560)])

    pltpu.sync_copy(src_hbm.at[pl.ds(w * NWIN, NWIN)], src2d)
    pltpu.sync_copy(dst_hbm.at[pl.ds(w * NWIN, NWIN)], dst2d)
    plsc.subcore_barrier()

    def hloop(h, _):
        pltpu.sync_copy(as_hbm.at[h], as_ts)
        pltpu.sync_copy(ad_hbm.at[h], ad_ts)
        base = w * EPW

        def eloop(i, _):
            r = i // 8
            col = (i % 8) * 16
            sv = src2d[r, pl.ds(col, 16)]
            dv = dst2d[r, pl.ds(col, 16)]
            av = plsc.load_gather(as_ts, [sv]) + plsc.load_gather(ad_ts, [dv])
            av = jnp.maximum(av, 0.2 * av)
            exv = jnp.exp(av)
            gid = base + i * 16 + lax.iota(jnp.int32, 16)
            exv = jnp.where(gid < E, exv, 0.0)
            ex_ts[r, pl.ds(col, 16)] = exv
            eidx_ts[r, pl.ds(col, 16)] = dv + h * NP
            return 0
        lax.fori_loop(0, EPW // 16, eloop, 0)

        pltpu.sync_copy(ex_ts, ex_hbm.at[h, pl.ds(w * NWIN, NWIN)])
        for g in range(NWIN // 8):
            descs = [
                pltpu.async_copy(ex_ts.at[g * 8 + k],
                                 esum_sh.at[eidx_ts.at[g * 8 + k]],
                                 sem, add=True)
                for k in range(8)
            ]
            for d_ in descs:
                d_.wait()
        return 0
    lax.fori_loop(0, H, hloop, 0)

    plsc.subcore_barrier()
    pltpu.sync_copy(esum_sh.at[pl.ds(s * 2560, 2560)],
                    esum_hbm.at[c, pl.ds(s * 2560, 2560)])


def _k2(src2, dst2, a_s_t, a_d_t):
    return pl.kernel(
        _k2_body,
        out_type=[
            jax.ShapeDtypeStruct((H, EP // WN, WN), jnp.float32),
            jax.ShapeDtypeStruct((NC, H * NP), jnp.float32),
        ],
        mesh=_mesh(),
        compiler_params=pltpu.CompilerParams(needs_layout_passes=False),
        scratch_types=[
            pltpu.VMEM((NWIN, WN), jnp.int32),
            pltpu.VMEM((NWIN, WN), jnp.int32),
            pltpu.VMEM((N,), jnp.float32),
            pltpu.VMEM((N,), jnp.float32),
            pltpu.VMEM((NWIN, WN), jnp.float32),
            pltpu.VMEM((NWIN, WN), jnp.int32),
            pltpu.VMEM((2560,), jnp.float32),
            pltpu.VMEM_SHARED((H * NP,), jnp.float32),
            pltpu.SemaphoreType.DMA,
        ],
    )(src2, dst2, a_s_t, a_d_t)


def _k2b_body(ep_ref, o_ref):
    o_ref[...] = ep_ref[0] + ep_ref[1]


def _k2b(esum_part):
    ep3 = esum_part.reshape(NC, (H * NP) // 128, 128)
    out = pl.pallas_call(
        _k2b_body,
        out_shape=jax.ShapeDtypeStruct(((H * NP) // 128, 128), jnp.float32),
    )(ep3)
    return out


def _bcast16(vec, j):
    idx = jnp.full((16, 1), j, jnp.int32)
    return lax.gather(
        vec, idx,
        lax.GatherDimensionNumbers(offset_dims=(), collapsed_slice_dims=(0,),
                                   start_index_map=(0,)),
        (1,), mode=lax.GatherScatterMode.PROMISE_IN_BOUNDS)


def _k3_body(src_hbm, dst_hbm, xs_hbm, ex_hbm, esum_hbm, agg_hbm,
             gidx2d, dst2d, wv_ts, rbuf0, rbuf1, acc_sh,
             gsem0, gsem1, ssem0, ssem1):
    c = lax.axis_index("c")
    s = lax.axis_index("s")
    w = c * NS + s

    pltpu.sync_copy(src_hbm.at[pl.ds(w * NWIN, NWIN)], gidx2d)
    pltpu.sync_copy(dst_hbm.at[pl.ds(w * NWIN, NWIN)], dst2d)

    def scale(buf, wi):
        def gloop(g, _):
            w16 = wv_ts[wi, pl.ds(g * 16, 16)]
            for j in range(16):
                wb = _bcast16(w16, j)
                row = g * 16 + j
                for k in range(8):
                    ksl = pl.ds(k * 16, 16)
                    buf[row, ksl] = buf[row, ksl] * wb
            return 0
        lax.fori_loop(0, WN // 16, gloop, 0)

    def hloop(h, _):
        pltpu.sync_copy(esum_hbm.at[pl.ds(h * (NP // 128), NP // 128)],
                        rbuf0.at[pl.ds(0, NP // 128)])
        pltpu.sync_copy(ex_hbm.at[h, pl.ds(w * NWIN, NWIN)], wv_ts)

        def wloop(i, _):
            r = i // 8
            col = (i % 8) * 16
            sl = pl.ds(col, 16)
            dv = dst2d[r, sl]
            esv = plsc.load_gather(rbuf0, [dv >> 7, dv & 127])
            wv_ts[r, sl] = wv_ts[r, sl] / (esv + 1e-16)
            gidx2d[r, sl] = gidx2d[r, sl] + (
                jnp.int32(N) * (h > 0).astype(jnp.int32))
            return 0
        lax.fori_loop(0, EPW // 16, wloop, 0)

        def zloop(i, _):
            rbuf0[i // 8, pl.ds((i % 8) * 16, 16)] = jnp.zeros((16,),
                                                              jnp.float32)
            return 0
        lax.fori_loop(0, WN * 8, zloop, 0)
        for k in range(5):
            pltpu.sync_copy(rbuf0, acc_sh.at[pl.ds(s * 640 + k * WN, WN)])
        plsc.subcore_barrier()

        pltpu.async_copy(xs_hbm.at[gidx2d.at[0]], rbuf0, gsem0)

        def pair(pi, _):
            w0 = 2 * pi
            w1 = w0 + 1
            pltpu.make_async_copy(xs_hbm.at[gidx2d.at[w0]], rbuf0,
                                  gsem0).wait()
            d1 = pltpu.async_copy(xs_hbm.at[gidx2d.at[w1]], rbuf1, gsem1)
            scale(rbuf0, w0)
            s0 = pltpu.async_copy(rbuf0, acc_sh.at[dst2d.at[w0]], ssem0,
                                  add=True)
            d1.wait()
            scale(rbuf1, w1)
            s0.wait()
            nxt = jnp.where(pi < NPAIR - 1, w0 + 2, 0)
            pltpu.async_copy(xs_hbm.at[gidx2d.at[nxt]], rbuf0, gsem0)
            s1 = pltpu.async_copy(rbuf1, acc_sh.at[dst2d.at[w1]], ssem1,
                                  add=True)
            s1.wait()
            return 0
        lax.fori_loop(0, NPAIR, pair, 0)
        pltpu.make_async_copy(xs_hbm.at[gidx2d.at[0]], rbuf0, gsem0).wait()

        plsc.subcore_barrier()
        for k in range(5):
            pltpu.sync_copy(acc_sh.at[pl.ds(s * 640 + k * WN, WN)],
                            agg_hbm.at[c, h, pl.ds(s * 640 + k * WN, WN)])
        return 0

    lax.fori_loop(0, H, hloop, 0)


def _k3(src2, dst2, xs_flat, ex_buf, esum_tot):
    return pl.kernel(
        _k3_body,
        out_type=[jax.ShapeDtypeStruct((NC, H, NP, C), jnp.float32)],
        mesh=_mesh(),
        compiler_params=pltpu.CompilerParams(needs_layout_passes=False),
        scratch_types=[
            pltpu.VMEM((NWIN, WN), jnp.int32),
            pltpu.VMEM((NWIN, WN), jnp.int32),
            pltpu.VMEM((NWIN, WN), jnp.float32),
            pltpu.VMEM((WN, C), jnp.float32),
            pltpu.VMEM((WN, C), jnp.float32),
            pltpu.VMEM_SHARED((NP, C), jnp.float32),
            pltpu.SemaphoreType.DMA,
            pltpu.SemaphoreType.DMA,
            pltpu.SemaphoreType.DMA,
            pltpu.SemaphoreType.DMA,
        ],
    )(src2, dst2, xs_flat, ex_buf, esum_tot)


_BN4 = 2000

def _k4_body(agg_ref, b_ref, g_ref, be_ref, pw_ref, o_ref):
    sh = []
    tot = jnp.zeros((_BN4, 1), jnp.float32)
    for h in range(H):
        sl = slice(h * C, (h + 1) * C)
        v = agg_ref[0, h] + agg_ref[1, h] + b_ref[:, sl]
        sh.append(v)
        tot = tot + jnp.sum(v, axis=1, keepdims=True)
    mu = tot / HC
    var = jnp.zeros((_BN4, 1), jnp.float32)
    for h in range(H):
        d = sh[h] - mu
        var = var + jnp.sum(d * d, axis=1, keepdims=True)
    inv = 1.0 / jnp.sqrt(var / HC + 1e-5)
    for h in range(H):
        sl = slice(h * C, (h + 1) * C)
        y = (sh[h] - mu) * inv * g_ref[:, sl] + be_ref[:, sl]
        o_ref[:, sl] = jnp.where(y > 0, y, pw_ref[:, sl] * y)


def _k4(agg, b2, g2, be2, pw2):
    return pl.pallas_call(
        _k4_body,
        grid=(N // _BN4,),
        in_specs=[
            pl.BlockSpec((NC, H, _BN4, C), lambda i: (0, 0, i, 0)),
            pl.BlockSpec((1, HC), lambda i: (0, 0)),
            pl.BlockSpec((1, HC), lambda i: (0, 0)),
            pl.BlockSpec((1, HC), lambda i: (0, 0)),
            pl.BlockSpec((1, HC), lambda i: (0, 0)),
        ],
        out_specs=pl.BlockSpec((_BN4, HC), lambda i: (i, 0)),
        out_shape=jax.ShapeDtypeStruct((N, HC), jnp.float32),
    )(agg, b2, g2, be2, pw2)


def kernel(x, edge_attr, edge_index, batch, W_src, W_dst, att_src, att_dst,
           bias, ln_gamma, ln_beta, prelu_w):
    src = edge_index[0]
    dst = edge_index[1]
    pad = (jnp.arange(EP - E, dtype=jnp.int32) * 37) % N
    src2 = jnp.concatenate([src, pad]).reshape(EP // WN, WN)
    dst2 = jnp.concatenate([dst, pad]).reshape(EP // WN, WN)

    att_s2 = att_src.reshape(1, HC)
    att_d2 = att_dst.reshape(1, HC)

    a_src, a_dst = _k1a(x, W_src, W_dst, att_s2, att_d2)
    a_s_t = a_src.T.reshape(H, N)
    a_d_t = a_dst.T.reshape(H, N)

    ex_buf, esum_part = _k2(src2, dst2, a_s_t, a_d_t)
    xs3 = _k1b(x, W_src)
    xs_flat = xs3.reshape(H * N, C)
    esum_tot = _k2b(esum_part)
    (agg,) = _k3(src2, dst2, xs_flat, ex_buf, esum_tot)

    b2 = bias.reshape(1, HC)
    g2 = ln_gamma.reshape(1, HC)
    be2 = ln_beta.reshape(1, HC)
    pw2 = prelu_w.reshape(1, HC)
    return _k4(agg, b2, g2, be2, pw2)
